# baseline (speedup 1.0000x reference)
import functools

import jax
import jax.numpy as jnp
from jax import lax
from jax.experimental import pallas as pl
from jax.experimental.pallas import tpu as pltpu
from jax.experimental.pallas import tpu_sc as plsc

_VD = 23040
_NROWS = 16
_B = 2
_C = 8
_N = 128 * 128 * 90 * 2
_W = 180
_H = _N // _W
_NW = 32
_R = 2
_NG = _NROWS // _R
_SUB_PER_G = _NW // _NG
_HPS = _H // _SUB_PER_G
_HL = 64
_K = _HL * _W
_CHUNKS = _HPS // _HL
_PITCH = 65
_TWO23 = 8388608.0
_GOFFS = tuple(range(0, 176, 16)) + (164,)


def _sc_gather(x_flat, idx):
    mesh = plsc.VectorSubcoreMesh(core_axis_name="c", subcore_axis_name="s")

    @functools.partial(
        pl.kernel,
        out_type=jax.ShapeDtypeStruct((_B, _W, _C, _H), jnp.float32),
        mesh=mesh,
        compiler_params=pltpu.CompilerParams(
            needs_layout_passes=False, use_tc_tiling_on_sc=False
        ),
        scratch_types=[
            pltpu.VMEM((_R * _VD,), jnp.float32),
            pltpu.VMEM((2, _K), jnp.float32),
            pltpu.VMEM((2, _R, _W, _PITCH), jnp.float32),
            pltpu.SemaphoreType.DMA,
            pltpu.SemaphoreType.DMA,
            pltpu.SemaphoreType.DMA,
            pltpu.SemaphoreType.DMA,
            pltpu.SemaphoreType.DMA,
        ],
    )
    def k(x_hbm, idx_hbm, out_t, tab_v, ibuf, obuf,
          s_tab, s_in0, s_in1, s_out0, s_out1):
        s_in = (s_in0, s_in1)
        s_out = (s_out0, s_out1)
        cid = lax.axis_index("c")
        sid = lax.axis_index("s")
        wid = sid * 2 + cid
        g = wid // _SUB_PER_G
        slot = wid % _SUB_PER_G
        h_base = slot * _HPS
        base = h_base * _W

        tab_cp = pltpu.async_copy(
            x_hbm.at[pl.ds(g * (_R * _VD), _R * _VD)], tab_v, s_tab
        )
        pltpu.async_copy(idx_hbm.at[pl.ds(base, _K)], ibuf.at[0], s_in[0])
        tab_cp.wait()

        lanes = lax.iota(jnp.int32, 16)
        wvecs = [lanes + o for o in _GOFFS]

        def pair_body(h, carry):
            for b in range(2):
                c = h * 2 + b
                off = base + c * _K
                h0 = h_base + c * _HL

                @pl.when(c + 1 < _CHUNKS)
                def _():
                    pltpu.async_copy(
                        idx_hbm.at[pl.ds(off + _K, _K)],
                        ibuf.at[1 - b],
                        s_in[1 - b],
                    )

                pltpu.make_async_copy(
                    idx_hbm.at[pl.ds(off, _K)], ibuf.at[b], s_in[b]
                ).wait()

                @pl.when(c >= 2)
                def _():
                    for r in range(_R):
                        row = g * _R + r
                        pltpu.make_async_copy(
                            obuf.at[b, r, :, pl.ds(0, _HL)],
                            out_t.at[
                                row // _C, :, row % _C,
                                pl.ds(h0 - 2 * _HL, _HL),
                            ],
                            s_out[b],
                        ).wait()

                @plsc.parallel_loop(0, _HL, unroll=2)
                def _(j):
                    jo = j * _W
                    hv = jnp.full((16,), 0, jnp.int32) + j
                    for gi, o in enumerate(_GOFFS):
                        f = ibuf[b, pl.ds(jo + o, 16)]
                        zi = plsc.bitcast(f + _TWO23, jnp.int32)
                        ii = jnp.minimum(zi & 0x7FFFFF, _VD - 1)
                        for r in range(_R):
                            vals = plsc.load_gather(tab_v, [ii + r * _VD])
                            plsc.store_scatter(
                                obuf.at[b, r], [wvecs[gi], hv], vals
                            )

                for r in range(_R):
                    row = g * _R + r
                    pltpu.async_copy(
                        obuf.at[b, r, :, pl.ds(0, _HL)],
                        out_t.at[row // _C, :, row % _C, pl.ds(h0, _HL)],
                        s_out[b],
                    )
            return carry

        lax.fori_loop(0, _CHUNKS // 2, pair_body, 0)

        for c in (_CHUNKS - 2, _CHUNKS - 1):
            b = c % 2
            h0 = h_base + c * _HL
            for r in range(_R):
                row = g * _R + r
                pltpu.make_async_copy(
                    obuf.at[b, r, :, pl.ds(0, _HL)],
                    out_t.at[row // _C, :, row % _C, pl.ds(h0, _HL)],
                    s_out[b],
                ).wait()

    return k(x_flat, idx)


def kernel(input, indices):
    x_flat = input.reshape(_NROWS * _VD)
    out_t = _sc_gather(x_flat, indices)
    return jnp.transpose(out_t, (0, 2, 3, 1))


# speedup vs baseline: 1.0279x; 1.0279x over previous
import functools

import jax
import jax.numpy as jnp
from jax import lax
from jax.experimental import pallas as pl
from jax.experimental.pallas import tpu as pltpu
from jax.experimental.pallas import tpu_sc as plsc

_VD = 23040
_NROWS = 16
_B = 2
_C = 8
_N = 128 * 128 * 90 * 2
_W = 180
_H = _N // _W
_NW = 32
_HL = 128
_K = _HL * _W
_HPS = _H // 2
_CHUNKS = _HPS // _HL
_PITCH = 129
_TWO23 = 8388608.0
_GOFFS = tuple(range(0, 176, 16)) + (164,)


def _sc_gather(x_flat, idx):
    mesh = plsc.VectorSubcoreMesh(core_axis_name="c", subcore_axis_name="s")

    @functools.partial(
        pl.kernel,
        out_type=jax.ShapeDtypeStruct((_B, _W, _C, _H), jnp.float32),
        mesh=mesh,
        compiler_params=pltpu.CompilerParams(
            needs_layout_passes=False, use_tc_tiling_on_sc=False
        ),
        scratch_types=[
            pltpu.VMEM((_VD,), jnp.float32),
            pltpu.VMEM((2, _K), jnp.float32),
            pltpu.VMEM((2, _W, _PITCH), jnp.float32),
            pltpu.SemaphoreType.DMA,
            pltpu.SemaphoreType.DMA,
            pltpu.SemaphoreType.DMA,
            pltpu.SemaphoreType.DMA,
            pltpu.SemaphoreType.DMA,
        ],
    )
    def k(x_hbm, idx_hbm, out_t, tab_v, ibuf, obuf,
          s_tab, s_in0, s_in1, s_out0, s_out1):
        s_in = (s_in0, s_in1)
        s_out = (s_out0, s_out1)
        cid = lax.axis_index("c")
        sid = lax.axis_index("s")
        wid = sid * 2 + cid
        bc = wid // 2
        half = wid % 2
        bi = bc // _C
        ci = bc % _C
        h_base = half * _HPS
        base = h_base * _W

        tab_cp = pltpu.async_copy(
            x_hbm.at[pl.ds(bc * _VD, _VD)], tab_v, s_tab
        )
        pltpu.async_copy(idx_hbm.at[pl.ds(base, _K)], ibuf.at[0], s_in[0])
        tab_cp.wait()

        lanes = lax.iota(jnp.int32, 16)
        wvecs = [lanes + o for o in _GOFFS]

        def pair_body(h, carry):
            for b in range(2):
                c = h * 2 + b
                off = base + c * _K
                h0 = h_base + c * _HL

                @pl.when(c + 1 < _CHUNKS)
                def _():
                    pltpu.async_copy(
                        idx_hbm.at[pl.ds(off + _K, _K)],
                        ibuf.at[1 - b],
                        s_in[1 - b],
                    )

                pltpu.make_async_copy(
                    idx_hbm.at[pl.ds(off, _K)], ibuf.at[b], s_in[b]
                ).wait()

                @pl.when(c >= 2)
                def _():
                    pltpu.make_async_copy(
                        obuf.at[b, :, pl.ds(0, _HL)],
                        out_t.at[bi, :, ci, pl.ds(h0 - 2 * _HL, _HL)],
                        s_out[b],
                    ).wait()

                @plsc.parallel_loop(0, _HL, unroll=2)
                def _(j):
                    jo = j * _W
                    hv = jnp.full((16,), 0, jnp.int32) + j
                    for gi, o in enumerate(_GOFFS):
                        f = ibuf[b, pl.ds(jo + o, 16)]
                        zi = plsc.bitcast(f + _TWO23, jnp.int32)
                        ii = jnp.minimum(zi & 0x7FFFFF, _VD - 1)
                        vals = plsc.load_gather(tab_v, [ii])
                        plsc.store_scatter(
                            obuf.at[b], [wvecs[gi], hv], vals
                        )

                pltpu.async_copy(
                    obuf.at[b, :, pl.ds(0, _HL)],
                    out_t.at[bi, :, ci, pl.ds(h0, _HL)],
                    s_out[b],
                )
            return carry

        lax.fori_loop(0, _CHUNKS // 2, pair_body, 0)

        for c in (_CHUNKS - 2, _CHUNKS - 1):
            b = c % 2
            h0 = h_base + c * _HL
            pltpu.make_async_copy(
                obuf.at[b, :, pl.ds(0, _HL)],
                out_t.at[bi, :, ci, pl.ds(h0, _HL)],
                s_out[b],
            ).wait()

    return k(x_flat, idx)


def kernel(input, indices):
    x_flat = input.reshape(_NROWS * _VD)
    out_t = _sc_gather(x_flat, indices)
    return jnp.transpose(out_t, (0, 2, 3, 1))
